# Initial kernel scaffold; baseline (speedup 1.0000x reference)
import jax
import jax.numpy as jnp
from jax.experimental import pallas as pl


def kernel(node_feat, mask_node, g0_w, g0_b, g1_w, g1_b, g2_w, g2_b, wb0, g3_w, g3_b, wb1, samp_key):
    raise NotImplementedError("write your pallas kernel here")


# speedup vs baseline: 1.0771x; 1.0771x over previous
import math

import jax
import jax.numpy as jnp
from jax.experimental import pallas as pl
from jax.experimental.pallas import tpu as pltpu

_EPS = 1e-10
_FILT = 0.7


def _diag_mask(n):
    r = jax.lax.broadcasted_iota(jnp.int32, (n, n), 0)
    c = jax.lax.broadcasted_iota(jnp.int32, (n, n), 1)
    return r == c


def _attention(adj, h, m, wb):
    n = adj.shape[0]
    att = jnp.dot(h, wb, preferred_element_type=jnp.float32)
    att = att + (m - 1.0) * 1e10
    e = jnp.exp(att - jnp.max(att, axis=0, keepdims=True))
    denom = jnp.dot(adj, e, preferred_element_type=jnp.float32) + _EPS
    dm = _diag_mask(n)
    diag_a = jnp.sum(jnp.where(dm, adj, 0.0), axis=1, keepdims=True)
    rowsum = jnp.sum(adj, axis=1, keepdims=True)
    return e * diag_a / denom * rowsum * m


def _lane_dense(v, n):
    return jnp.sum(jnp.where(_diag_mask(n), v, 0.0), axis=0, keepdims=True)


def _block0_body(x_ref, m_ref, w0_ref, b0_ref, w1_ref, b1_ref, w2_ref,
                 b2_ref, wb_ref, adj_ref, att_ref, z_ref):
    x = x_ref[0]
    m = m_ref[0]
    n = x.shape[0]

    nrm = jnp.sqrt(jnp.sum(x * x, axis=-1, keepdims=True))
    xn = x / jnp.maximum(nrm, 1e-12)
    a = jax.lax.dot_general(xn, xn, (((1,), (1,)), ((), ())),
                            preferred_element_type=jnp.float32)
    a = 0.5 * jnp.tanh(a) + 0.5
    deg_c = jnp.sum(a, axis=1, keepdims=True)
    deg_c = jnp.where(deg_c == 0.0, 1e-10, deg_c)
    deg_r = jnp.sum(a, axis=0, keepdims=True)
    deg_r = jnp.where(deg_r == 0.0, 1e-10, deg_r)
    adj = jax.lax.rsqrt(deg_c) * a * jax.lax.rsqrt(deg_r)
    adj_ref[0] = adj

    h = xn
    for w, b in ((w0_ref, b0_ref), (w1_ref, b1_ref), (w2_ref, b2_ref)):
        y = jnp.dot(adj, h, preferred_element_type=jnp.float32)
        y = jnp.dot(y, w[...], preferred_element_type=jnp.float32) + b[...]
        h = jnp.maximum(y, 0.0)
    h = m * h

    att_b = _attention(adj, h, m, wb_ref[...])
    z_ref[0] = att_b * h
    att_ref[0] = _lane_dense(att_b, n)


def _pool_gcn_body(idx_ref, val_ref, z_ref, adj_ref, w_ref, b_ref, wb_ref,
                   nadj_ref, att_ref, z1_ref):
    idx = idx_ref[0]
    val = val_ref[0]
    z = z_ref[0]
    a = adj_ref[0]
    k = idx.shape[0]
    n = a.shape[0]

    cols = jax.lax.broadcasted_iota(jnp.int32, (k, n), 1)
    sel = jnp.where(cols == idx, val, 0.0)
    assign = jnp.dot(sel, a, preferred_element_type=jnp.float32)
    colsum = jnp.sum(assign, axis=0, keepdims=True)
    sn = assign / (colsum + _EPS)
    feat = jnp.dot(sn, z, preferred_element_type=jnp.float32)
    tmp = jnp.dot(sn, a, preferred_element_type=jnp.float32)
    nadj = jax.lax.dot_general(tmp, sn, (((1,), (1,)), ((), ())),
                               preferred_element_type=jnp.float32)
    nadj_ref[0] = nadj

    y = jnp.dot(nadj, feat, preferred_element_type=jnp.float32)
    y = jnp.dot(y, w_ref[...], preferred_element_type=jnp.float32) + b_ref[...]
    h = jnp.maximum(y, 0.0)
    h = val * h

    att_b = _attention(nadj, h, val, wb_ref[...])
    z1_ref[0] = att_b * h
    att_ref[0] = _lane_dense(att_b, k)


def _pool_final_body(idx_ref, val_ref, z_ref, adj_ref, h_ref):
    idx = idx_ref[0]
    val = val_ref[0]
    z = z_ref[0]
    a = adj_ref[0]
    k = idx.shape[0]
    n = a.shape[0]

    cols = jax.lax.broadcasted_iota(jnp.int32, (k, n), 1)
    sel = jnp.where(cols == idx, val, 0.0)
    assign = jnp.dot(sel, a, preferred_element_type=jnp.float32)
    colsum = jnp.sum(assign, axis=0, keepdims=True)
    sn = assign / (colsum + _EPS)
    h_ref[0] = jnp.dot(sn, z, preferred_element_type=jnp.float32)


def _whole(shape):
    nd = len(shape)
    return pl.BlockSpec((1,) + shape[1:], lambda b: (b,) + (0,) * (nd - 1))


def _bcast(arr):
    return pl.BlockSpec(arr.shape, lambda b: (0,) * arr.ndim)


_PAR = pltpu.CompilerParams(dimension_semantics=("parallel",))


def _sample(att, mask, key):
    bsz, n = mask.shape
    k_max = int(math.ceil(_FILT * n))
    k_list = jnp.ceil(_FILT * jnp.sum(mask, axis=1)).astype(jnp.int32)
    p = att * mask
    p = p / (jnp.sum(p, axis=1, keepdims=True) + _EPS)
    logits = jnp.log(p + 1e-30)
    keys = jax.random.split(key, bsz)
    top_index = jax.vmap(
        lambda k_, lg: jax.random.categorical(k_, lg, shape=(k_max,)))(keys, logits)
    new_mask = (jax.lax.broadcasted_iota(jnp.int32, (bsz, k_max), 1)
                < k_list[:, None]).astype(jnp.float32)
    return top_index.astype(jnp.int32).reshape(bsz, k_max, 1), new_mask


def kernel(node_feat, mask_node, g0_w, g0_b, g1_w, g1_b, g2_w, g2_b,
           wb0, g3_w, g3_b, wb1, samp_key):
    bsz, n, _ = node_feat.shape
    hid = g0_w.shape[1]
    k0 = int(math.ceil(_FILT * n))
    k1 = int(math.ceil(_FILT * k0))

    key = jax.random.key(samp_key)
    keys = jax.random.split(key, 2)

    adj, att0, z0 = pl.pallas_call(
        _block0_body,
        out_shape=(jax.ShapeDtypeStruct((bsz, n, n), jnp.float32),
                   jax.ShapeDtypeStruct((bsz, 1, n), jnp.float32),
                   jax.ShapeDtypeStruct((bsz, n, hid), jnp.float32)),
        grid=(bsz,),
        in_specs=[_whole((bsz, n, node_feat.shape[2])),
                  _whole((bsz, n, 1)),
                  _bcast(g0_w), _bcast(g0_b), _bcast(g1_w), _bcast(g1_b),
                  _bcast(g2_w), _bcast(g2_b), _bcast(wb0)],
        out_specs=(_whole((bsz, n, n)), _whole((bsz, 1, n)),
                   _whole((bsz, n, hid))),
        compiler_params=_PAR,
    )(node_feat, mask_node.reshape(bsz, n, 1), g0_w, g0_b, g1_w, g1_b,
      g2_w, g2_b, wb0)

    idx0, mask1 = _sample(att0.reshape(bsz, n), mask_node, keys[0])

    nadj, att1, z1 = pl.pallas_call(
        _pool_gcn_body,
        out_shape=(jax.ShapeDtypeStruct((bsz, k0, k0), jnp.float32),
                   jax.ShapeDtypeStruct((bsz, 1, k0), jnp.float32),
                   jax.ShapeDtypeStruct((bsz, k0, hid), jnp.float32)),
        grid=(bsz,),
        in_specs=[_whole((bsz, k0, 1)), _whole((bsz, k0, 1)),
                  _whole((bsz, n, hid)), _whole((bsz, n, n)),
                  _bcast(g3_w), _bcast(g3_b), _bcast(wb1)],
        out_specs=(_whole((bsz, k0, k0)), _whole((bsz, 1, k0)),
                   _whole((bsz, k0, hid))),
        compiler_params=_PAR,
    )(idx0, mask1.reshape(bsz, k0, 1), z0, adj, g3_w, g3_b, wb1)

    idx1, mask2 = _sample(att1.reshape(bsz, k0), mask1, keys[1])

    x_out = pl.pallas_call(
        _pool_final_body,
        out_shape=jax.ShapeDtypeStruct((bsz, k1, hid), jnp.float32),
        grid=(bsz,),
        in_specs=[_whole((bsz, k1, 1)), _whole((bsz, k1, 1)),
                  _whole((bsz, k0, hid)), _whole((bsz, k0, k0))],
        out_specs=_whole((bsz, k1, hid)),
        compiler_params=_PAR,
    )(idx1, mask2.reshape(bsz, k1, 1), z1, nadj)

    return x_out
